# baseline (speedup 1.0000x reference)
import jax
import jax.numpy as jnp
from jax.experimental import pallas as pl
from jax.experimental.pallas import tpu as pltpu

EPS = 1e-12
NEG_BIG = -1e30
CHUNK = 128


def _ema_kernel(ptr2_ref, z_ref, out_ref):
    L, D = z_ref.shape[1], z_ref.shape[2]
    T = CHUNK
    NC = L // T

    rid = jax.lax.broadcasted_iota(jnp.int32, (T, T), 0)
    cid = jax.lax.broadcasted_iota(jnp.int32, (T, T), 1)
    tril = rid >= cid
    tril_f = tril.astype(jnp.float32)
    triu_f = (rid <= cid).astype(jnp.float32)

    for g in range(z_ref.shape[0]):
        ptr2 = ptr2_ref[g]
        ldr = jnp.log(jnp.maximum(1.0 - ptr2, EPS))
        u_col = jax.lax.dot_general(
            tril_f, ldr, (((1,), (1,)), ((), ())),
            precision=jax.lax.Precision.HIGHEST,
            preferred_element_type=jnp.float32)
        u_row = jax.lax.dot_general(
            ldr, triu_f, (((1,), (0,)), ((), ())),
            precision=jax.lax.Precision.HIGHEST,
            preferred_element_type=jnp.float32)

        z = z_ref[g]
        c = jnp.zeros((1, D), jnp.float32)
        for k in range(NC):
            r = k * T
            sc = u_col[:, k:k + 1]
            sr = u_row[k:k + 1, :]
            ptr = ptr2[k:k + 1, :]
            delta = jnp.where(tril, sc - sr, NEG_BIG)
            w = jnp.exp(delta) * ptr
            f = jnp.exp(sc - sc[0:1, :])
            out_c = jax.lax.dot_general(
                w.astype(jnp.bfloat16), z[r:r + T],
                (((1,), (0,)), ((), ())),
                preferred_element_type=jnp.float32) + f * c
            out_ref[g, r:r + T, :] = out_c
            if k + 1 < NC:
                dec_next = jnp.maximum(1.0 - ptr2[k + 1:k + 2, 0:1], EPS)
                c = dec_next * out_c[T - 1:T]


@jax.jit
def kernel(z, pt):
    B, L, D = z.shape
    T = CHUNK
    NC = L // T
    pt_row2 = pt.reshape(B, NC, T)
    z = z.astype(jnp.bfloat16)
    GB = 2
    out = pl.pallas_call(
        _ema_kernel,
        grid=(B // GB,),
        in_specs=[
            pl.BlockSpec((GB, NC, T), lambda b: (b, 0, 0)),
            pl.BlockSpec((GB, L, D), lambda b: (b, 0, 0)),
        ],
        out_specs=pl.BlockSpec((GB, L, D), lambda b: (b, 0, 0)),
        out_shape=jax.ShapeDtypeStruct((B, L, D), jnp.float32),
        compiler_params=pltpu.CompilerParams(
            dimension_semantics=("parallel",)),
    )(pt_row2, z)
    return out


# speedup vs baseline: 1.4921x; 1.4921x over previous
import jax
import jax.numpy as jnp
from jax.experimental import pallas as pl
from jax.experimental.pallas import tpu as pltpu

EPS = 1e-12
NEG_BIG = -1e30
CHUNK = 64


def _ema_kernel(ptr2_ref, z_ref, out_ref):
    L, D = z_ref.shape[1], z_ref.shape[2]
    T = CHUNK
    NC = L // T

    rid = jax.lax.broadcasted_iota(jnp.int32, (T, T), 0)
    cid = jax.lax.broadcasted_iota(jnp.int32, (T, T), 1)
    tril = rid >= cid
    tril_f = tril.astype(jnp.float32)
    triu_f = (rid <= cid).astype(jnp.float32)

    for g in range(z_ref.shape[0]):
        ptr2 = ptr2_ref[g]
        ldr = jnp.log(jnp.maximum(1.0 - ptr2, EPS))
        u_col = jax.lax.dot_general(
            tril_f, ldr, (((1,), (1,)), ((), ())),
            precision=jax.lax.Precision.HIGHEST,
            preferred_element_type=jnp.float32)
        u_row = jax.lax.dot_general(
            ldr, triu_f, (((1,), (0,)), ((), ())),
            precision=jax.lax.Precision.HIGHEST,
            preferred_element_type=jnp.float32)

        z = z_ref[g]
        c = jnp.zeros((1, D), jnp.float32)
        for k in range(NC):
            r = k * T
            sc = u_col[:, k:k + 1]
            sr = u_row[k:k + 1, :]
            ptr = ptr2[k:k + 1, :]
            delta = jnp.where(tril, sc - sr, NEG_BIG)
            w = jnp.exp(delta) * ptr
            f = jnp.exp(sc - sc[0:1, :])
            out_c = jax.lax.dot_general(
                w.astype(jnp.bfloat16), z[r:r + T],
                (((1,), (0,)), ((), ())),
                preferred_element_type=jnp.float32) + f * c
            out_ref[g, r:r + T, :] = out_c
            if k + 1 < NC:
                dec_next = jnp.maximum(1.0 - ptr2[k + 1:k + 2, 0:1], EPS)
                c = dec_next * out_c[T - 1:T]


@jax.jit
def kernel(z, pt):
    B, L, D = z.shape
    T = CHUNK
    NC = L // T
    pt_row2 = pt.reshape(B, NC, T)
    GB = 2
    out = pl.pallas_call(
        _ema_kernel,
        grid=(B // GB,),
        in_specs=[
            pl.BlockSpec((GB, NC, T), lambda b: (b, 0, 0)),
            pl.BlockSpec((GB, L, D), lambda b: (b, 0, 0)),
        ],
        out_specs=pl.BlockSpec((GB, L, D), lambda b: (b, 0, 0)),
        out_shape=jax.ShapeDtypeStruct((B, L, D), jnp.float32),
        compiler_params=pltpu.CompilerParams(
            dimension_semantics=("parallel",)),
    )(pt_row2, z)
    return out


# speedup vs baseline: 1.5464x; 1.0364x over previous
import jax
import jax.numpy as jnp
from jax.experimental import pallas as pl
from jax.experimental.pallas import tpu as pltpu

EPS = 1e-12
NEG_BIG = -1e30
CHUNK = 64


def _ema_kernel(ptr2_ref, z_ref, out_ref):
    L, D = z_ref.shape[1], z_ref.shape[2]
    T = CHUNK
    NC = L // T

    rid = jax.lax.broadcasted_iota(jnp.int32, (T, T), 0)
    cid = jax.lax.broadcasted_iota(jnp.int32, (T, T), 1)
    tril = rid >= cid
    tril_f = tril.astype(jnp.float32)
    triu_f = (rid <= cid).astype(jnp.float32)

    for g in range(z_ref.shape[0]):
        ptr2 = ptr2_ref[g]
        ldr = jnp.log(jnp.maximum(1.0 - ptr2, EPS))
        u_col = jax.lax.dot_general(
            tril_f, ldr, (((1,), (1,)), ((), ())),
            precision=jax.lax.Precision.HIGHEST,
            preferred_element_type=jnp.float32)
        u_row = jax.lax.dot_general(
            ldr, triu_f, (((1,), (0,)), ((), ())),
            precision=jax.lax.Precision.HIGHEST,
            preferred_element_type=jnp.float32)

        z = z_ref[g]
        c = jnp.zeros((1, D), jnp.float32)
        for k in range(NC):
            r = k * T
            sc = u_col[:, k:k + 1]
            sr = u_row[k:k + 1, :]
            ptr = ptr2[k:k + 1, :]
            delta = jnp.where(tril, sc - sr, NEG_BIG)
            w = jnp.exp(delta) * ptr
            f = jnp.exp(sc - sc[0:1, :])
            out_c = jax.lax.dot_general(
                w.astype(jnp.bfloat16), z[r:r + T],
                (((1,), (0,)), ((), ())),
                preferred_element_type=jnp.float32) + f * c
            out_ref[g, r:r + T, :] = out_c
            if k + 1 < NC:
                dec_next = jnp.maximum(1.0 - ptr2[k + 1:k + 2, 0:1], EPS)
                c = dec_next * out_c[T - 1:T]


@jax.jit
def kernel(z, pt):
    B, L, D = z.shape
    T = CHUNK
    NC = L // T
    pt_row2 = pt.reshape(B, NC, T)
    GB = 4
    out = pl.pallas_call(
        _ema_kernel,
        grid=(B // GB,),
        in_specs=[
            pl.BlockSpec((GB, NC, T), lambda b: (b, 0, 0)),
            pl.BlockSpec((GB, L, D), lambda b: (b, 0, 0)),
        ],
        out_specs=pl.BlockSpec((GB, L, D), lambda b: (b, 0, 0)),
        out_shape=jax.ShapeDtypeStruct((B, L, D), jnp.float32),
        compiler_params=pltpu.CompilerParams(
            dimension_semantics=("parallel",)),
    )(pt_row2, z)
    return out
